# baseline (speedup 1.0000x reference)
import jax
import jax.numpy as jnp
from jax import lax
from jax.experimental import pallas as pl
from jax.experimental.pallas import tpu as pltpu
from jax.experimental.pallas import tpu_sc as plsc

RES = 128
FEAT = 8
LANES = 16

NC = 2
NS = 16
NW = NC * NS

PAD_N = 1 << 21
NPW = PAD_N // NW
CHUNK = 512
NCHUNK = NPW // CHUNK
SUB = 128
NSUB = CHUNK // SUB

CORNER_OFF = tuple((dx * RES + dy) * RES + dz
                   for dx in (0, 1) for dy in (0, 1) for dz in (0, 1))


def _tec_body(pts_hbm, feat_hbm, out_hbm,
              pts_v, fx_v, fy_v, fz_v, idx_v, corner_v, out_v, gsem):
    cid = lax.axis_index("c")
    sid = lax.axis_index("s")
    wid = sid * NC + cid
    wbase = wid * NPW

    iota = lax.iota(jnp.int32, LANES)
    bsel = iota >> 3
    feat_lane = iota & 7
    zeros16 = jnp.zeros((LANES,), jnp.int32)
    ones16 = zeros16 + 1
    twos16 = zeros16 + 2

    def chunk_body(t, _):
        base = wbase + t * CHUNK
        pltpu.sync_copy(pts_hbm.at[pl.ds(base, CHUNK), :], pts_v)

        def grp(g, _):
            rows = g * LANES + iota
            px = plsc.load_gather(pts_v, [rows, zeros16])
            py = plsc.load_gather(pts_v, [rows, ones16])
            pz = plsc.load_gather(pts_v, [rows, twos16])

            def split(p):
                x = (p + 1.0) * (0.5 * (RES - 1))
                xi = jnp.clip(x.astype(jnp.int32), 0, RES - 2)
                return xi, x - xi.astype(jnp.float32)

            xi, fx = split(px)
            yi, fy = split(py)
            zi, fz = split(pz)
            off = g * LANES
            fx_v[pl.ds(off, LANES)] = fx
            fy_v[pl.ds(off, LANES)] = fy
            fz_v[pl.ds(off, LANES)] = fz
            flat = (xi * RES + yi) * RES + zi
            j = g >> 3
            o = (g & 7) * LANES
            for c in range(8):
                idx_v[c, j, pl.ds(o, LANES)] = flat + CORNER_OFF[c]
            return 0

        lax.fori_loop(0, CHUNK // LANES, grp, 0, unroll=2)

        copies = []
        for c in range(8):
            for j in range(NSUB):
                copies.append(pltpu.async_copy(
                    feat_hbm.at[idx_v.at[c, j]],
                    corner_v.at[c, pl.ds(j * SUB, SUB), :],
                    gsem))
        for cp in copies:
            cp.wait()

        def pair(p, _):
            rows2 = 2 * p + bsel
            fxb = plsc.load_gather(fx_v, [rows2])
            fyb = plsc.load_gather(fy_v, [rows2])
            fzb = plsc.load_gather(fz_v, [rows2])
            wx = (1.0 - fxb, fxb)
            wy = (1.0 - fyb, fyb)
            wz = (1.0 - fzb, fzb)
            acc = jnp.zeros((LANES,), jnp.float32)
            c = 0
            for dx in (0, 1):
                for dy in (0, 1):
                    wxy = wx[dx] * wy[dy]
                    for dz in (0, 1):
                        w = wxy * wz[dz]
                        corner = plsc.load_gather(
                            corner_v, [zeros16 + c, rows2, feat_lane])
                        acc = acc + corner * w
                        c += 1
            out_v[pl.ds(p * LANES, LANES)] = acc
            return 0

        lax.fori_loop(0, CHUNK // 2, pair, 0, unroll=2)

        pltpu.sync_copy(out_v, out_hbm.at[pl.ds(base * FEAT, CHUNK * FEAT)])
        return 0

    lax.fori_loop(0, NCHUNK, chunk_body, 0)


@jax.jit
def _lod_interp(pts_pad, features):
    mesh = plsc.VectorSubcoreMesh(core_axis_name="c", subcore_axis_name="s")
    run = pl.kernel(
        _tec_body,
        out_type=jax.ShapeDtypeStruct((PAD_N * FEAT,), jnp.float32),
        mesh=mesh,
        compiler_params=pltpu.CompilerParams(
            needs_layout_passes=False,
            use_tc_tiling_on_sc=False,
        ),
        scratch_types=[
            pltpu.VMEM((CHUNK, 3), jnp.float32),
            pltpu.VMEM((CHUNK,), jnp.float32),
            pltpu.VMEM((CHUNK,), jnp.float32),
            pltpu.VMEM((CHUNK,), jnp.float32),
            pltpu.VMEM((8, NSUB, SUB), jnp.int32),
            pltpu.VMEM((8, CHUNK, FEAT), jnp.float32),
            pltpu.VMEM((CHUNK * FEAT,), jnp.float32),
            pltpu.SemaphoreType.DMA,
        ],
    )
    return run(pts_pad, features)


def kernel(pts, features):
    n = pts.shape[0]
    pts_pad = jnp.pad(pts, ((0, PAD_N - n), (0, 0)))
    out = _lod_interp(pts_pad, features)
    return out.reshape(PAD_N, FEAT)[:n]


# speedup vs baseline: 1.1394x; 1.1394x over previous
import jax
import jax.numpy as jnp
from jax import lax
from jax.experimental import pallas as pl
from jax.experimental.pallas import tpu as pltpu
from jax.experimental.pallas import tpu_sc as plsc

RES = 128
FEAT = 8
LANES = 16

NC = 2
NS = 16
NW = NC * NS

NPTS = 2_000_000
CHUNK = 512
NCHUNK_TOT = (NPTS + CHUNK - 1) // CHUNK
NCW = (NCHUNK_TOT + NW - 1) // NW
SUB = 128
NSUB = CHUNK // SUB

CORNER_OFF = tuple((dx * RES + dy) * RES + dz
                   for dx in (0, 1) for dy in (0, 1) for dz in (0, 1))


def _tec_body(pts_hbm, feat_hbm, out_hbm,
              pts_v, fx_v, fy_v, fz_v, idx_v, corner_v, out_v, gsem):
    cid = lax.axis_index("c")
    sid = lax.axis_index("s")
    wid = sid * NC + cid

    iota = lax.iota(jnp.int32, LANES)
    bsel = iota >> 3
    feat_lane = iota & 7
    zeros16 = jnp.zeros((LANES,), jnp.int32)
    ones16 = zeros16 + 1
    twos16 = zeros16 + 2

    def chunk_body(t, _):
        g = t * NW + wid

        @pl.when(g < NCHUNK_TOT)
        def _():
            base = jnp.minimum(g * CHUNK, NPTS - CHUNK)
            pltpu.sync_copy(pts_hbm.at[pl.ds(base * 3, CHUNK * 3)], pts_v)

            def grp(gg, _):
                rows3 = (gg * LANES + iota) * 3
                px = plsc.load_gather(pts_v, [rows3])
                py = plsc.load_gather(pts_v, [rows3 + 1])
                pz = plsc.load_gather(pts_v, [rows3 + 2])

                def split(p):
                    x = (p + 1.0) * (0.5 * (RES - 1))
                    xi = jnp.clip(x.astype(jnp.int32), 0, RES - 2)
                    return xi, x - xi.astype(jnp.float32)

                xi, fx = split(px)
                yi, fy = split(py)
                zi, fz = split(pz)
                off = gg * LANES
                fx_v[pl.ds(off, LANES)] = fx
                fy_v[pl.ds(off, LANES)] = fy
                fz_v[pl.ds(off, LANES)] = fz
                flat = (xi * RES + yi) * RES + zi
                j = gg >> 3
                o = (gg & 7) * LANES
                for c in range(8):
                    idx_v[c, j, pl.ds(o, LANES)] = flat + CORNER_OFF[c]
                return 0

            lax.fori_loop(0, CHUNK // LANES, grp, 0, unroll=2)

            copies = []
            for c in range(8):
                for j in range(NSUB):
                    copies.append(pltpu.async_copy(
                        feat_hbm.at[idx_v.at[c, j]],
                        corner_v.at[c, pl.ds(j * SUB, SUB), :],
                        gsem))
            for cp in copies:
                cp.wait()

            def pair(p, _):
                rows2 = 2 * p + bsel
                fxb = plsc.load_gather(fx_v, [rows2])
                fyb = plsc.load_gather(fy_v, [rows2])
                fzb = plsc.load_gather(fz_v, [rows2])
                wx = (1.0 - fxb, fxb)
                wy = (1.0 - fyb, fyb)
                wz = (1.0 - fzb, fzb)
                acc = jnp.zeros((LANES,), jnp.float32)
                c = 0
                for dx in (0, 1):
                    for dy in (0, 1):
                        wxy = wx[dx] * wy[dy]
                        for dz in (0, 1):
                            w = wxy * wz[dz]
                            corner = plsc.load_gather(
                                corner_v, [zeros16 + c, rows2, feat_lane])
                            acc = acc + corner * w
                            c += 1
                out_v[pl.ds(p * LANES, LANES)] = acc
                return 0

            lax.fori_loop(0, CHUNK // 2, pair, 0, unroll=2)

            pltpu.sync_copy(out_v, out_hbm.at[pl.ds(base * FEAT, CHUNK * FEAT)])

        return 0

    lax.fori_loop(0, NCW, chunk_body, 0)


@jax.jit
def _lod_interp(pts_flat, features):
    mesh = plsc.VectorSubcoreMesh(core_axis_name="c", subcore_axis_name="s")
    run = pl.kernel(
        _tec_body,
        out_type=jax.ShapeDtypeStruct((NPTS * FEAT,), jnp.float32),
        mesh=mesh,
        compiler_params=pltpu.CompilerParams(
            needs_layout_passes=False,
            use_tc_tiling_on_sc=False,
        ),
        scratch_types=[
            pltpu.VMEM((CHUNK * 3,), jnp.float32),
            pltpu.VMEM((CHUNK,), jnp.float32),
            pltpu.VMEM((CHUNK,), jnp.float32),
            pltpu.VMEM((CHUNK,), jnp.float32),
            pltpu.VMEM((8, NSUB, SUB), jnp.int32),
            pltpu.VMEM((8, CHUNK, FEAT), jnp.float32),
            pltpu.VMEM((CHUNK * FEAT,), jnp.float32),
            pltpu.SemaphoreType.DMA,
        ],
    )
    return run(pts_flat, features)


def kernel(pts, features):
    n = pts.shape[0]
    out = _lod_interp(pts.reshape(-1), features)
    return out.reshape(n, FEAT)


# speedup vs baseline: 1.1463x; 1.0061x over previous
import jax
import jax.numpy as jnp
from jax import lax
from jax.experimental import pallas as pl
from jax.experimental.pallas import tpu as pltpu
from jax.experimental.pallas import tpu_sc as plsc

RES = 128
FEAT = 8
LANES = 16

NC = 2
NS = 16
NW = NC * NS

NPTS = 2_000_000
CHUNK = 512
NCHUNK_TOT = (NPTS + CHUNK - 1) // CHUNK
NCW = (NCHUNK_TOT + NW - 1) // NW
SUB = 128
NSUB = CHUNK // SUB

CORNER_OFF = tuple((dx * RES + dy) * RES + dz
                   for dx in (0, 1) for dy in (0, 1) for dz in (0, 1))


def _tec_body(pts_hbm, feat_hbm, out_hbm,
              pts_v, fx_v, fy_v, fz_v, idx_v, corner_v, out_v, gsem):
    cid = lax.axis_index("c")
    sid = lax.axis_index("s")
    wid = sid * NC + cid

    iota = lax.iota(jnp.int32, LANES)
    bsel = iota >> 3
    feat_lane = iota & 7
    zeros16 = jnp.zeros((LANES,), jnp.int32)
    ones16 = zeros16 + 1
    twos16 = zeros16 + 2

    def chunk_body(t, _):
        g = t * NW + wid

        @pl.when(g < NCHUNK_TOT)
        def _():
            base = jnp.minimum(g * CHUNK, NPTS - CHUNK)
            pltpu.sync_copy(
                pts_hbm.at[pl.ds(base * 3 // 128, CHUNK * 3 // 128), :], pts_v)

            def grp(gg, _):
                rows3 = (gg * LANES + iota) * 3
                px = plsc.load_gather(pts_v, [rows3 >> 7, rows3 & 127])
                py = plsc.load_gather(pts_v, [(rows3 + 1) >> 7, (rows3 + 1) & 127])
                pz = plsc.load_gather(pts_v, [(rows3 + 2) >> 7, (rows3 + 2) & 127])

                def split(p):
                    x = (p + 1.0) * (0.5 * (RES - 1))
                    xi = jnp.clip(x.astype(jnp.int32), 0, RES - 2)
                    return xi, x - xi.astype(jnp.float32)

                xi, fx = split(px)
                yi, fy = split(py)
                zi, fz = split(pz)
                off = gg * LANES
                fx_v[pl.ds(off, LANES)] = fx
                fy_v[pl.ds(off, LANES)] = fy
                fz_v[pl.ds(off, LANES)] = fz
                flat = (xi * RES + yi) * RES + zi
                j = gg >> 3
                o = (gg & 7) * LANES
                for c in range(8):
                    idx_v[c, j, pl.ds(o, LANES)] = flat + CORNER_OFF[c]
                return 0

            lax.fori_loop(0, CHUNK // LANES, grp, 0, unroll=2)

            copies = []
            for c in range(8):
                for j in range(NSUB):
                    copies.append(pltpu.async_copy(
                        feat_hbm.at[idx_v.at[c, j]],
                        corner_v.at[c, pl.ds(j * SUB, SUB), :],
                        gsem))
            for cp in copies:
                cp.wait()

            def pair(p, _):
                rows2 = 2 * p + bsel
                fxb = plsc.load_gather(fx_v, [rows2])
                fyb = plsc.load_gather(fy_v, [rows2])
                fzb = plsc.load_gather(fz_v, [rows2])
                wx = (1.0 - fxb, fxb)
                wy = (1.0 - fyb, fyb)
                wz = (1.0 - fzb, fzb)
                acc = jnp.zeros((LANES,), jnp.float32)
                c = 0
                for dx in (0, 1):
                    for dy in (0, 1):
                        wxy = wx[dx] * wy[dy]
                        for dz in (0, 1):
                            w = wxy * wz[dz]
                            corner = plsc.load_gather(
                                corner_v, [zeros16 + c, rows2, feat_lane])
                            acc = acc + corner * w
                            c += 1
                out_v[p >> 3, pl.ds((p & 7) * LANES, LANES)] = acc
                return 0

            lax.fori_loop(0, CHUNK // 2, pair, 0, unroll=2)

            pltpu.sync_copy(
                out_v,
                out_hbm.at[pl.ds(base * FEAT // 128, CHUNK * FEAT // 128), :])

        return 0

    lax.fori_loop(0, NCW, chunk_body, 0)


@jax.jit
def _lod_interp(pts_flat, features):
    mesh = plsc.VectorSubcoreMesh(core_axis_name="c", subcore_axis_name="s")
    run = pl.kernel(
        _tec_body,
        out_type=jax.ShapeDtypeStruct((NPTS * FEAT // 128, 128), jnp.float32),
        mesh=mesh,
        compiler_params=pltpu.CompilerParams(
            needs_layout_passes=False,
            use_tc_tiling_on_sc=False,
        ),
        scratch_types=[
            pltpu.VMEM((CHUNK * 3 // 128, 128), jnp.float32),
            pltpu.VMEM((CHUNK,), jnp.float32),
            pltpu.VMEM((CHUNK,), jnp.float32),
            pltpu.VMEM((CHUNK,), jnp.float32),
            pltpu.VMEM((8, NSUB, SUB), jnp.int32),
            pltpu.VMEM((8, CHUNK, FEAT), jnp.float32),
            pltpu.VMEM((CHUNK * FEAT // 128, 128), jnp.float32),
            pltpu.SemaphoreType.DMA,
        ],
    )
    return run(pts_flat, features)


def kernel(pts, features):
    n = pts.shape[0]
    out = _lod_interp(pts.reshape(n * 3 // 128, 128), features)
    return out.reshape(n, FEAT)
